# baseline (speedup 1.0000x reference)
import functools
import jax
import jax.numpy as jnp
from jax import lax
from jax.experimental import pallas as pl
from jax.experimental.pallas import tpu as pltpu
from jax.experimental.pallas import tpu_sc as plsc

S = 200
B = 4096
E = 64
EW = E // 2
VOCAB = 1000000
NPOS = 201
NW = 32
BW = B // NW
NBUF = 10
LA = 6


def _make_sc_gather():
    mesh = plsc.VectorSubcoreMesh(core_axis_name="c", subcore_axis_name="s")

    @functools.partial(
        pl.kernel,
        mesh=mesh,
        out_type=jax.ShapeDtypeStruct((S * B, EW), jnp.int32),
        compiler_params=pltpu.CompilerParams(use_tc_tiling_on_sc=False),
        scratch_types=[
            pltpu.VMEM((S, BW), jnp.int32),
        ]
        + [pltpu.VMEM((BW, EW), jnp.int32) for _ in range(NBUF)]
        + [pltpu.SemaphoreType.DMA for _ in range(2 * NBUF)],
    )
    def k(idx_hbm, table_hbm, out_hbm, idx_v, *bufsem):
        bufs = bufsem[:NBUF]
        gsems = bufsem[NBUF:2 * NBUF]
        wsems = bufsem[2 * NBUF:]
        nc = lax.axis_index("c")
        ns = lax.axis_index("s")
        wid = ns * 2 + nc

        pltpu.sync_copy(idx_hbm.at[wid], idx_v)

        def gather_start(s, kb):
            pltpu.make_async_copy(
                table_hbm.at[idx_v.at[s]], bufs[kb], gsems[kb]).start()

        def gather_wait(kb):
            pltpu.make_async_copy(
                table_hbm.at[idx_v.at[0]], bufs[kb], gsems[kb]).wait()

        def wb_start(s, kb):
            pltpu.make_async_copy(
                bufs[kb], out_hbm.at[pl.ds(s * B + wid * BW, BW)],
                wsems[kb]).start()

        def wb_wait(kb):
            pltpu.make_async_copy(
                bufs[kb], out_hbm.at[pl.ds(wid * BW, BW)], wsems[kb]).wait()

        for s0 in range(LA):
            gather_start(s0, s0)

        def g_body(g, _):
            for kb in range(NBUF):
                s = NBUF * g + kb
                gather_wait(kb)
                wb_start(s, kb)

                k2 = (kb + LA) % NBUF
                s2 = s + LA

                @pl.when(s2 < S)
                def _():
                    @pl.when(s2 >= NBUF)
                    def _():
                        wb_wait(k2)
                    gather_start(s2, k2)
            return 0

        lax.fori_loop(0, S // NBUF, g_body, 0)
        for kb in range(NBUF):
            wb_wait(kb)

    return k


_sc_gather = _make_sc_gather()


def _tc_add_body(raw_ref, pos_ref, out_ref):
    s = pl.program_id(0)
    w = raw_ref[0]
    lo = lax.bitcast_convert_type(w << 16, jnp.float32)
    hi = lax.bitcast_convert_type(
        w & jnp.int32(-65536), jnp.float32)
    pos_lo = pos_ref[pl.ds(s, 1), pl.ds(0, EW)]
    pos_hi = pos_ref[pl.ds(s, 1), pl.ds(EW, EW)]
    out_ref[0, :, pl.ds(0, EW)] = lo + pos_lo
    out_ref[0, :, pl.ds(EW, EW)] = hi + pos_hi


_tc_add = pl.pallas_call(
    _tc_add_body,
    grid=(S,),
    in_specs=[
        pl.BlockSpec((1, B, EW), lambda s: (s, 0, 0)),
        pl.BlockSpec((NPOS, E), lambda s: (0, 0)),
    ],
    out_specs=pl.BlockSpec((1, B, E), lambda s: (s, 0, 0)),
    out_shape=jax.ShapeDtypeStruct((S, B, E), jnp.float32),
)


def kernel(word_seq, word_emb, pos_table, word_pos):
    idx = jnp.transpose(word_seq.reshape(S, NW, BW), (1, 0, 2))
    u = lax.bitcast_convert_type(word_emb, jnp.uint32)
    r = (u + jnp.uint32(0x7FFF) + ((u >> 16) & jnp.uint32(1))) >> 16
    w = r[:, :EW] | (r[:, EW:] << 16)
    tbl = lax.bitcast_convert_type(w, jnp.int32)
    raw = _sc_gather(idx, tbl)
    return _tc_add(raw.reshape(S, B, EW), pos_table)


# speedup vs baseline: 2.0075x; 2.0075x over previous
import functools
import jax
import jax.numpy as jnp
from jax import lax
from jax.experimental import pallas as pl
from jax.experimental.pallas import tpu as pltpu
from jax.experimental.pallas import tpu_sc as plsc

S = 200
B = 4096
E = 64
NPOS = 201
NW = 32
BW = B // NW
LANES = 16
NBUF = 10
LA = 6
RUNROLL = 4


def _make_kernel():
    mesh = plsc.VectorSubcoreMesh(core_axis_name="c", subcore_axis_name="s")

    @functools.partial(
        pl.kernel,
        mesh=mesh,
        out_type=jax.ShapeDtypeStruct((S * B, E), jnp.float32),
        compiler_params=pltpu.CompilerParams(use_tc_tiling_on_sc=False),
        scratch_types=[
            pltpu.VMEM((S, BW), jnp.int32),
            pltpu.VMEM((NPOS * E,), jnp.float32),
        ]
        + [pltpu.VMEM((BW, E), jnp.float32) for _ in range(NBUF)]
        + [pltpu.SemaphoreType.DMA for _ in range(2 * NBUF)],
    )
    def k(idx_hbm, table_hbm, pos_hbm, out_hbm, idx_v, pos_v, *bufsem):
        bufs = bufsem[:NBUF]
        gsems = bufsem[NBUF:2 * NBUF]
        wsems = bufsem[2 * NBUF:]
        nc = lax.axis_index("c")
        ns = lax.axis_index("s")
        wid = ns * 2 + nc

        pltpu.sync_copy(pos_hbm, pos_v)
        pltpu.sync_copy(idx_hbm.at[wid], idx_v)

        def gather_start(s, kb):
            pltpu.make_async_copy(
                table_hbm.at[idx_v.at[s]], bufs[kb], gsems[kb]).start()

        def gather_wait(kb):
            pltpu.make_async_copy(
                table_hbm.at[idx_v.at[0]], bufs[kb], gsems[kb]).wait()

        def wb_start(s, kb):
            pltpu.make_async_copy(
                bufs[kb], out_hbm.at[pl.ds(s * B + wid * BW, BW)],
                wsems[kb]).start()

        def wb_wait(kb):
            pltpu.make_async_copy(
                bufs[kb], out_hbm.at[pl.ds(wid * BW, BW)], wsems[kb]).wait()

        for s0 in range(LA):
            gather_start(s0, s0)

        def g_body(g, _):
            for kb in range(NBUF):
                s = NBUF * g + kb
                gather_wait(kb)

                p = [pos_v[pl.ds(s * E + j * LANES, LANES)]
                     for j in range(E // LANES)]

                buf = bufs[kb]

                def r_body(r, _):
                    for rr in range(RUNROLL):
                        row = RUNROLL * r + rr
                        for j in range(E // LANES):
                            plsc.addupdate(
                                buf.at[row, pl.ds(j * LANES, LANES)], p[j])
                    return 0

                lax.fori_loop(0, BW // RUNROLL, r_body, 0)

                wb_start(s, kb)

                k2 = (kb + LA) % NBUF
                s2 = s + LA

                @pl.when(s2 < S)
                def _():
                    @pl.when(s2 >= NBUF)
                    def _():
                        wb_wait(k2)
                    gather_start(s2, k2)
            return 0

        lax.fori_loop(0, S // NBUF, g_body, 0)
        for kb in range(NBUF):
            wb_wait(kb)

    return k


_sc_kernel = _make_kernel()


def kernel(word_seq, word_emb, pos_table, word_pos):
    idx = jnp.transpose(word_seq.reshape(S, NW, BW), (1, 0, 2))
    pos_flat = pos_table.reshape(NPOS * E)
    out = _sc_kernel(idx, word_emb, pos_flat)
    return out.reshape(S, B, E)


# speedup vs baseline: 2.0078x; 1.0002x over previous
import functools
import jax
import jax.numpy as jnp
from jax import lax
from jax.experimental import pallas as pl
from jax.experimental.pallas import tpu as pltpu
from jax.experimental.pallas import tpu_sc as plsc

S = 200
B = 4096
E = 64
NPOS = 201
NW = 32
BW = B // NW
LANES = 16
NBUF = 10
LA = 8
RUNROLL = 4


def _make_kernel():
    mesh = plsc.VectorSubcoreMesh(core_axis_name="c", subcore_axis_name="s")

    @functools.partial(
        pl.kernel,
        mesh=mesh,
        out_type=jax.ShapeDtypeStruct((S * B, E), jnp.float32),
        compiler_params=pltpu.CompilerParams(use_tc_tiling_on_sc=False),
        scratch_types=[
            pltpu.VMEM((S, BW), jnp.int32),
            pltpu.VMEM((NPOS * E,), jnp.float32),
        ]
        + [pltpu.VMEM((BW, E), jnp.float32) for _ in range(NBUF)]
        + [pltpu.SemaphoreType.DMA for _ in range(2 * NBUF)],
    )
    def k(idx_hbm, table_hbm, pos_hbm, out_hbm, idx_v, pos_v, *bufsem):
        bufs = bufsem[:NBUF]
        gsems = bufsem[NBUF:2 * NBUF]
        wsems = bufsem[2 * NBUF:]
        nc = lax.axis_index("c")
        ns = lax.axis_index("s")
        wid = ns * 2 + nc

        pltpu.sync_copy(pos_hbm, pos_v)
        pltpu.sync_copy(idx_hbm.at[wid], idx_v)

        def gather_start(s, kb):
            pltpu.make_async_copy(
                table_hbm.at[idx_v.at[s]], bufs[kb], gsems[kb]).start()

        def gather_wait(kb):
            pltpu.make_async_copy(
                table_hbm.at[idx_v.at[0]], bufs[kb], gsems[kb]).wait()

        def wb_start(s, kb):
            pltpu.make_async_copy(
                bufs[kb], out_hbm.at[pl.ds(s * B + wid * BW, BW)],
                wsems[kb]).start()

        def wb_wait(kb):
            pltpu.make_async_copy(
                bufs[kb], out_hbm.at[pl.ds(wid * BW, BW)], wsems[kb]).wait()

        for s0 in range(LA):
            gather_start(s0, s0)

        def g_body(g, _):
            for kb in range(NBUF):
                s = NBUF * g + kb
                gather_wait(kb)

                p = [pos_v[pl.ds(s * E + j * LANES, LANES)]
                     for j in range(E // LANES)]

                buf = bufs[kb]

                def r_body(r, _):
                    for rr in range(RUNROLL):
                        row = RUNROLL * r + rr
                        for j in range(E // LANES):
                            plsc.addupdate(
                                buf.at[row, pl.ds(j * LANES, LANES)], p[j])
                    return 0

                lax.fori_loop(0, BW // RUNROLL, r_body, 0)

                wb_start(s, kb)

                k2 = (kb + LA) % NBUF
                s2 = s + LA

                @pl.when(s2 < S)
                def _():
                    @pl.when(s2 >= NBUF)
                    def _():
                        wb_wait(k2)
                    gather_start(s2, k2)
            return 0

        lax.fori_loop(0, S // NBUF, g_body, 0)
        for kb in range(NBUF):
            wb_wait(kb)

    return k


_sc_kernel = _make_kernel()


def kernel(word_seq, word_emb, pos_table, word_pos):
    idx = jnp.transpose(word_seq.reshape(S, NW, BW), (1, 0, 2))
    pos_flat = pos_table.reshape(NPOS * E)
    out = _sc_kernel(idx, word_emb, pos_flat)
    return out.reshape(S, B, E)
